# baseline (device time: 34700 ns/iter reference)
import jax
import jax.numpy as jnp
from jax import lax
from jax.experimental import pallas as pl
from jax.experimental.pallas import tpu as pltpu

BLOCKS = 8


def kernel(x):
    m, n = x.shape
    mb = m // BLOCKS

    def body(x_ref, out_ref, send_buf, recv_buf, send_sem, recv_sem):
        i = pl.program_id(0)
        my_x = lax.axis_index("x")
        my_y = lax.axis_index("y")
        nbr = (my_x, 1 - my_y)

        send_buf[pl.ds(i * mb, mb), :] = jnp.max(
            x_ref[...], axis=1, keepdims=True
        )

        @pl.when(i == BLOCKS - 1)
        def _():
            barrier_sem = pltpu.get_barrier_semaphore()
            pl.semaphore_signal(
                barrier_sem, inc=1,
                device_id=nbr, device_id_type=pl.DeviceIdType.MESH,
            )
            pl.semaphore_wait(barrier_sem, 1)

            rdma = pltpu.make_async_remote_copy(
                src_ref=send_buf,
                dst_ref=recv_buf,
                send_sem=send_sem,
                recv_sem=recv_sem,
                device_id=nbr,
                device_id_type=pl.DeviceIdType.MESH,
            )
            rdma.start()
            rdma.wait()

            out_ref[...] = jnp.maximum(send_buf[...], recv_buf[...])

    return pl.pallas_call(
        body,
        grid=(BLOCKS,),
        out_shape=jax.ShapeDtypeStruct((m, 1), x.dtype),
        in_specs=[
            pl.BlockSpec((mb, n), lambda i: (i, 0), memory_space=pltpu.VMEM)
        ],
        out_specs=pl.BlockSpec((m, 1), lambda i: (0, 0), memory_space=pltpu.VMEM),
        scratch_shapes=[
            pltpu.VMEM((m, 1), x.dtype),
            pltpu.VMEM((m, 1), x.dtype),
            pltpu.SemaphoreType.DMA,
            pltpu.SemaphoreType.DMA,
        ],
        compiler_params=pltpu.CompilerParams(collective_id=0),
    )(x)


# device time: 11625 ns/iter; 2.9849x vs baseline; 2.9849x over previous
import jax
import jax.numpy as jnp
from jax import lax
from jax.experimental import pallas as pl
from jax.experimental.pallas import tpu as pltpu

BLOCKS = 8
LANES = 128


def kernel(x):
    m, n = x.shape
    mb = m // BLOCKS
    pk = m // LANES

    def body(x_ref, out_ref, send_buf, recv_buf, send_sem, recv_sem):
        i = pl.program_id(0)
        my_x = lax.axis_index("x")
        my_y = lax.axis_index("y")
        nbr = (my_x, 1 - my_y)

        blk = jnp.max(x_ref[...], axis=1)
        send_buf[pl.ds(i * (mb // LANES), mb // LANES), :] = blk.reshape(
            mb // LANES, LANES
        )

        @pl.when(i == BLOCKS - 1)
        def _():
            barrier_sem = pltpu.get_barrier_semaphore()
            pl.semaphore_signal(
                barrier_sem, inc=1,
                device_id=nbr, device_id_type=pl.DeviceIdType.MESH,
            )
            pl.semaphore_wait(barrier_sem, 1)

            rdma = pltpu.make_async_remote_copy(
                src_ref=send_buf,
                dst_ref=recv_buf,
                send_sem=send_sem,
                recv_sem=recv_sem,
                device_id=nbr,
                device_id_type=pl.DeviceIdType.MESH,
            )
            rdma.start()
            rdma.wait()

            out_ref[...] = jnp.maximum(send_buf[...], recv_buf[...])

    packed = pl.pallas_call(
        body,
        grid=(BLOCKS,),
        out_shape=jax.ShapeDtypeStruct((pk, LANES), x.dtype),
        in_specs=[
            pl.BlockSpec((mb, n), lambda i: (i, 0), memory_space=pltpu.VMEM)
        ],
        out_specs=pl.BlockSpec(
            (pk, LANES), lambda i: (0, 0), memory_space=pltpu.VMEM
        ),
        scratch_shapes=[
            pltpu.VMEM((pk, LANES), x.dtype),
            pltpu.VMEM((pk, LANES), x.dtype),
            pltpu.SemaphoreType.DMA,
            pltpu.SemaphoreType.DMA,
        ],
        compiler_params=pltpu.CompilerParams(collective_id=0),
    )(x)
    return packed.reshape(m, 1)


# device time: 11608 ns/iter; 2.9893x vs baseline; 1.0015x over previous
import jax
import jax.numpy as jnp
from jax import lax
from jax.experimental import pallas as pl
from jax.experimental.pallas import tpu as pltpu

BLOCKS = 8
LANES = 128


def kernel(x):
    m, n = x.shape
    mb = m // BLOCKS
    pk = m // LANES
    pb = pk // BLOCKS
    half = pk // 2

    def body(x_ref, out_ref, send_buf, recv_buf, send_sems, recv_sems):
        i = pl.program_id(0)
        my_x = lax.axis_index("x")
        my_y = lax.axis_index("y")
        nbr = (my_x, 1 - my_y)

        def half_rdma(h):
            return pltpu.make_async_remote_copy(
                src_ref=send_buf.at[pl.ds(h * half, half), :],
                dst_ref=recv_buf.at[pl.ds(h * half, half), :],
                send_sem=send_sems.at[h],
                recv_sem=recv_sems.at[h],
                device_id=nbr,
                device_id_type=pl.DeviceIdType.MESH,
            )

        barrier_sem = pltpu.get_barrier_semaphore()

        @pl.when(i == 0)
        def _():
            pl.semaphore_signal(
                barrier_sem, inc=1,
                device_id=nbr, device_id_type=pl.DeviceIdType.MESH,
            )

        blk = jnp.max(x_ref[...], axis=1)
        send_buf[pl.ds(i * pb, pb), :] = blk.reshape(pb, LANES)

        @pl.when(i == BLOCKS // 2 - 1)
        def _():
            pl.semaphore_wait(barrier_sem, 1)
            half_rdma(0).start()

        @pl.when(i == BLOCKS - 1)
        def _():
            half_rdma(1).start()
            r0 = half_rdma(0)
            r1 = half_rdma(1)
            r0.wait_send()
            r0.wait_recv()
            r1.wait_send()
            r1.wait_recv()
            out_ref[...] = jnp.maximum(send_buf[...], recv_buf[...])

    packed = pl.pallas_call(
        body,
        grid=(BLOCKS,),
        out_shape=jax.ShapeDtypeStruct((pk, LANES), x.dtype),
        in_specs=[
            pl.BlockSpec((mb, n), lambda i: (i, 0), memory_space=pltpu.VMEM)
        ],
        out_specs=pl.BlockSpec(
            (pk, LANES), lambda i: (0, 0), memory_space=pltpu.VMEM
        ),
        scratch_shapes=[
            pltpu.VMEM((pk, LANES), x.dtype),
            pltpu.VMEM((pk, LANES), x.dtype),
            pltpu.SemaphoreType.DMA((2,)),
            pltpu.SemaphoreType.DMA((2,)),
        ],
        compiler_params=pltpu.CompilerParams(collective_id=0),
    )(x)
    return packed.reshape(m, 1)
